# Initial kernel scaffold; baseline (speedup 1.0000x reference)
import jax
import jax.numpy as jnp
from jax.experimental import pallas as pl


def kernel(hidden_states, sim_matrix, threshold, w1, w2):
    raise NotImplementedError("write your pallas kernel here")


# speedup vs baseline: 1.9033x; 1.9033x over previous
import jax
import jax.numpy as jnp
from jax.experimental import pallas as pl
from jax.experimental.pallas import tpu as pltpu

_HIDDEN = 1024
_EXPERTS = 8
_INTER = 4096


def _gate_kernel(x_ref, sim_ref, thr_ref, scores_ref, rw_ref, k_ref):
    x = x_ref[...]
    sim = sim_ref[...]
    nx = x / jnp.maximum(jnp.sqrt(jnp.sum(x * x, axis=1, keepdims=True)), 1e-12)
    nsim = sim / jnp.maximum(jnp.sqrt(jnp.sum(sim * sim, axis=0, keepdims=True)), 1e-12)
    scores = jax.lax.dot_general(nx, nsim, (((1,), (0,)), ((), ())),
                                 preferred_element_type=jnp.float32)
    thr = thr_ref[0, 0]
    mask = scores > thr
    k = jnp.sum(mask.astype(jnp.int32), axis=1, keepdims=True)
    ms = jnp.where(mask, scores, -1e9)
    m = jnp.max(ms, axis=1, keepdims=True)
    ew = jnp.exp(ms - m)
    rw = ew / jnp.sum(ew, axis=1, keepdims=True)
    scores_ref[...] = scores
    rw_ref[...] = rw
    k_ref[...] = k


def _ffn_kernel(x_ref, w1_ref, w2_ref, rw_ref, out_ref):
    e = pl.program_id(0)
    i = pl.program_id(1)

    x = x_ref[...]
    w1b = w1_ref[0].astype(jnp.bfloat16)
    h = jax.lax.dot_general(x, w1b, (((1,), (1,)), ((), ())),
                            preferred_element_type=jnp.float32)
    h = 0.5 * h * (1.0 + jax.lax.erf(h * 0.7071067811865476))
    hb = h.astype(jnp.bfloat16)
    w2b = w2_ref[0].astype(jnp.bfloat16)
    part = jax.lax.dot_general(hb, w2b, (((1,), (1,)), ((), ())),
                               preferred_element_type=jnp.float32)

    rw = rw_ref[...]
    lane = jax.lax.broadcasted_iota(jnp.int32, rw.shape, 1)
    w = jnp.sum(jnp.where(lane == e, rw, 0.0), axis=1, keepdims=True)
    contrib = part * w

    @pl.when(jnp.logical_and(e == 0, i == 0))
    def _init():
        out_ref[...] = contrib

    @pl.when(jnp.logical_or(e != 0, i != 0))
    def _acc():
        out_ref[...] += contrib


def kernel(hidden_states, sim_matrix, threshold, w1, w2):
    Bs, Ts, C = hidden_states.shape
    N = Bs * Ts
    x = hidden_states.reshape(N, C)

    BN_G = 512
    scores, rw, k2 = pl.pallas_call(
        _gate_kernel,
        grid=(N // BN_G,),
        in_specs=[
            pl.BlockSpec((BN_G, C), lambda r: (r, 0)),
            pl.BlockSpec((C, _EXPERTS), lambda r: (0, 0)),
            pl.BlockSpec((1, 1), lambda r: (0, 0)),
        ],
        out_specs=[
            pl.BlockSpec((BN_G, _EXPERTS), lambda r: (r, 0)),
            pl.BlockSpec((BN_G, _EXPERTS), lambda r: (r, 0)),
            pl.BlockSpec((BN_G, 1), lambda r: (r, 0)),
        ],
        out_shape=[
            jax.ShapeDtypeStruct((N, _EXPERTS), jnp.float32),
            jax.ShapeDtypeStruct((N, _EXPERTS), jnp.float32),
            jax.ShapeDtypeStruct((N, 1), jnp.int32),
        ],
    )(x, sim_matrix, threshold.reshape(1, 1))

    xb = x.astype(jnp.bfloat16)

    BI = 256
    NI = _INTER // BI
    final = pl.pallas_call(
        _ffn_kernel,
        grid=(_EXPERTS, NI),
        in_specs=[
            pl.BlockSpec((N, C), lambda e, i: (0, 0)),
            pl.BlockSpec((1, BI, _HIDDEN), lambda e, i: (e, i, 0)),
            pl.BlockSpec((1, _HIDDEN, BI), lambda e, i: (e, 0, i)),
            pl.BlockSpec((N, _EXPERTS), lambda e, i: (0, 0)),
        ],
        out_specs=pl.BlockSpec((N, C), lambda e, i: (0, 0)),
        out_shape=jax.ShapeDtypeStruct((N, C), jnp.float32),
        compiler_params=pltpu.CompilerParams(
            dimension_semantics=("arbitrary", "arbitrary"),
        ),
    )(xb, w1, w2, rw)

    return (final.reshape(Bs, Ts, C), scores, k2.reshape(N))
